# baseline (speedup 1.0000x reference)
import jax
import jax.numpy as jnp
from jax.experimental import pallas as pl
from jax.experimental.pallas import tpu as pltpu

_S = 4
_K = 8192
_D = 32
_CHUNK = 2048
_NCHUNK = _K // _CHUNK
_T = _S * _NCHUNK


def _tr_body(cb_ref, out_ref):
    out_ref[0] = cb_ref[0].T


def _rq_body(z_ref, cb_ref, cbt_ref, iota_ref, zq_ref, loss_ref,
             dbuf_ref, best_ref, qbest_ref, res_ref, recon_ref, lacc_ref):
    t = pl.program_id(0)
    p = t - 1

    @pl.when(t == 0)
    def _init():
        res_ref[...] = z_ref[...]
        recon_ref[...] = jnp.zeros_like(recon_ref)
        lacc_ref[...] = jnp.zeros_like(lacc_ref)

    @pl.when(t > 0)
    def _consume():
        dists = dbuf_ref[(t - 1) % 2]
        ib = iota_ref[...]
        nl = 128
        nj = _CHUNK // nl
        run_min = dists[:, 0:nl]
        run_idx = jnp.zeros_like(run_min)
        for j in range(1, nj):
            d_j = dists[:, j * nl:(j + 1) * nl]
            lt = d_j < run_min
            run_min = jnp.where(lt, d_j, run_min)
            run_idx = jnp.where(lt, float(j), run_idx)
        m = jnp.min(run_min, axis=1, keepdims=True)
        comb = run_idx * float(nl) + ib[:, 0:nl]
        li = jnp.min(jnp.where(run_min == m, comb, float(_CHUNK)),
                     axis=1, keepdims=True)
        onehot = (ib == li).astype(jnp.bfloat16)

        cb = cb_ref[0]
        hi = cb.astype(jnp.bfloat16)
        r1 = cb - hi.astype(jnp.float32)
        mid = r1.astype(jnp.bfloat16)
        lo = (r1 - mid.astype(jnp.float32)).astype(jnp.bfloat16)
        csplit = jnp.concatenate([hi, mid, lo], axis=1)
        qc3 = jax.lax.dot_general(
            onehot, csplit, (((1,), (0,)), ((), ())),
            preferred_element_type=jnp.float32)

        first = p % _NCHUNK == 0
        upd = jnp.logical_or(m < best_ref[...], first)
        best_ref[...] = jnp.where(upd, m, best_ref[...])
        qbest_ref[...] = jnp.where(upd, qc3, qbest_ref[...])

    @pl.when(jnp.logical_and(t > 0, p % _NCHUNK == _NCHUNK - 1))
    def _stage_fin():
        qb = qbest_ref[...]
        q = (qb[:, :_D] + qb[:, _D:2 * _D]) + qb[:, 2 * _D:]
        err = q - res_ref[...]
        sq = jnp.sum(err * err, axis=1, keepdims=True)
        lacc_ref[...] += jnp.sum(sq, axis=0, keepdims=True) / (err.shape[0] * _D)
        recon_ref[...] += q
        res_ref[...] = res_ref[...] - q

    @pl.when(t == _T)
    def _fin():
        z = z_ref[...]
        zq_ref[...] = z + (recon_ref[...] - z)
        loss_ref[...] = lacc_ref[...] * 1.25

    @pl.when(t < _T)
    def _produce():
        residual = res_ref[...]
        rb2 = (residual + residual).astype(jnp.bfloat16)
        lhs = jnp.concatenate(
            [rb2, jnp.ones((rb2.shape[0], 3), jnp.bfloat16)], axis=1)
        cbt = cbt_ref[0]
        c2 = jnp.sum(cbt * cbt, axis=0, keepdims=True)
        h1 = c2.astype(jnp.bfloat16)
        rr1 = c2 - h1.astype(jnp.float32)
        h2 = rr1.astype(jnp.bfloat16)
        h3 = (rr1 - h2.astype(jnp.float32)).astype(jnp.bfloat16)
        rhs = jnp.concatenate(
            [(-cbt).astype(jnp.bfloat16), h1, h2, h3], axis=0)
        dbuf_ref[t % 2] = jax.lax.dot_general(
            lhs, rhs, (((1,), (0,)), ((), ())),
            preferred_element_type=jnp.float32)


def kernel(z, codebooks):
    orig_shape = z.shape
    z_flat = z.reshape(-1, _D)
    n = z_flat.shape[0]
    cbt = pl.pallas_call(
        _tr_body,
        grid=(_S,),
        in_specs=[pl.BlockSpec((1, _K, _D), lambda s: (s, 0, 0))],
        out_specs=pl.BlockSpec((1, _D, _K), lambda s: (s, 0, 0)),
        out_shape=jax.ShapeDtypeStruct((_S, _D, _K), jnp.float32),
    )(codebooks)
    iota_row = jnp.arange(_CHUNK, dtype=jnp.float32).reshape(1, _CHUNK)
    zq, loss = pl.pallas_call(
        _rq_body,
        grid=(_T + 1,),
        in_specs=[
            pl.BlockSpec((n, _D), lambda t: (0, 0)),
            pl.BlockSpec((1, _CHUNK, _D),
                         lambda t: (jnp.maximum(t - 1, 0) // _NCHUNK,
                                    jnp.maximum(t - 1, 0) % _NCHUNK, 0)),
            pl.BlockSpec((1, _D, _CHUNK),
                         lambda t: (jnp.minimum(t, _T - 1) // _NCHUNK, 0,
                                    jnp.minimum(t, _T - 1) % _NCHUNK)),
            pl.BlockSpec((1, _CHUNK), lambda t: (0, 0)),
        ],
        out_specs=(
            pl.BlockSpec((n, _D), lambda t: (0, 0)),
            pl.BlockSpec((1, 1), lambda t: (0, 0)),
        ),
        out_shape=(
            jax.ShapeDtypeStruct((n, _D), jnp.float32),
            jax.ShapeDtypeStruct((1, 1), jnp.float32),
        ),
        scratch_shapes=[
            pltpu.VMEM((2, n, _CHUNK), jnp.float32),
            pltpu.VMEM((n, 1), jnp.float32),
            pltpu.VMEM((n, 3 * _D), jnp.float32),
            pltpu.VMEM((n, _D), jnp.float32),
            pltpu.VMEM((n, _D), jnp.float32),
            pltpu.VMEM((1, 1), jnp.float32),
        ],
    )(z_flat, codebooks, cbt, iota_row)
    return zq.reshape(orig_shape), loss[0, 0]


# speedup vs baseline: 1.1197x; 1.1197x over previous
import jax
import jax.numpy as jnp
from jax.experimental import pallas as pl
from jax.experimental.pallas import tpu as pltpu

_S = 4
_K = 8192
_D = 32
_CHUNK = 2048
_NCHUNK = _K // _CHUNK
_T = _S * _NCHUNK


def _tr_body(cb_ref, out_ref):
    out_ref[0] = cb_ref[0].T


def _rq_body(z_ref, cb_ref, cbt_ref, iota_ref, zq_ref, loss_ref,
             dbuf_ref, best_ref, qbest_ref, res_ref, recon_ref, lacc_ref):
    t = pl.program_id(0)
    p = t - 1

    @pl.when(t == 0)
    def _init():
        res_ref[...] = z_ref[...]
        recon_ref[...] = jnp.zeros_like(recon_ref)
        lacc_ref[...] = jnp.zeros_like(lacc_ref)

    @pl.when(t > 0)
    def _consume():
        dists = dbuf_ref[(t - 1) % 2]
        ib = iota_ref[...]
        nl = 128
        nj = _CHUNK // nl
        run_min = dists[:, 0:nl]
        run_idx = jnp.zeros_like(run_min)
        for j in range(1, nj):
            d_j = dists[:, j * nl:(j + 1) * nl]
            lt = d_j < run_min
            run_min = jnp.where(lt, d_j, run_min)
            run_idx = jnp.where(lt, float(j), run_idx)
        m = jnp.min(run_min, axis=1, keepdims=True)
        comb = run_idx * float(nl) + ib[:, 0:nl]
        li = jnp.min(jnp.where(run_min == m, comb, float(_CHUNK)),
                     axis=1, keepdims=True)
        onehot = (ib == li).astype(jnp.bfloat16)

        cb = cb_ref[0]
        hi = cb.astype(jnp.bfloat16)
        r1 = cb - hi.astype(jnp.float32)
        mid = r1.astype(jnp.bfloat16)
        lo = (r1 - mid.astype(jnp.float32)).astype(jnp.bfloat16)
        csplit = jnp.concatenate([hi, mid, lo], axis=1)
        qc3 = jax.lax.dot_general(
            onehot, csplit, (((1,), (0,)), ((), ())),
            preferred_element_type=jnp.float32)

        first = p % _NCHUNK == 0
        upd = jnp.logical_or(m < best_ref[...], first)
        best_ref[...] = jnp.where(upd, m, best_ref[...])
        qbest_ref[...] = jnp.where(upd, qc3, qbest_ref[...])

    @pl.when(jnp.logical_and(t > 0, p % _NCHUNK == _NCHUNK - 1))
    def _stage_fin():
        qb = qbest_ref[...]
        q = (qb[:, :_D] + qb[:, _D:2 * _D]) + qb[:, 2 * _D:]
        err = q - res_ref[...]
        sq = jnp.sum(err * err, axis=1, keepdims=True)
        lacc_ref[...] += jnp.sum(sq, axis=0, keepdims=True) / (err.shape[0] * _D)
        recon_ref[...] += q
        res_ref[...] = res_ref[...] - q

    @pl.when(t == _T)
    def _fin():
        z = z_ref[...]
        zq_ref[...] = z + (recon_ref[...] - z)
        loss_ref[...] = lacc_ref[...] * 1.25

    @pl.when(t < _T)
    def _produce():
        residual = res_ref[...]
        rb2 = (residual + residual).astype(jnp.bfloat16)
        lhs = jnp.concatenate(
            [rb2, jnp.ones((rb2.shape[0], 3), jnp.bfloat16)], axis=1)
        cbt = cbt_ref[0]
        c2 = jnp.sum(cbt * cbt, axis=0, keepdims=True)
        h1 = c2.astype(jnp.bfloat16)
        rr1 = c2 - h1.astype(jnp.float32)
        h2 = rr1.astype(jnp.bfloat16)
        h3 = (rr1 - h2.astype(jnp.float32)).astype(jnp.bfloat16)
        rhs = jnp.concatenate(
            [(-cbt).astype(jnp.bfloat16), h1, h2, h3], axis=0)
        dbuf_ref[t % 2] = jax.lax.dot_general(
            lhs, rhs, (((1,), (0,)), ((), ())),
            preferred_element_type=jnp.float32)


def kernel(z, codebooks):
    orig_shape = z.shape
    z_flat = z.reshape(-1, _D)
    n = z_flat.shape[0]
    cbt = codebooks.transpose(0, 2, 1)
    iota_row = jnp.arange(_CHUNK, dtype=jnp.float32).reshape(1, _CHUNK)
    zq, loss = pl.pallas_call(
        _rq_body,
        grid=(_T + 1,),
        in_specs=[
            pl.BlockSpec((n, _D), lambda t: (0, 0)),
            pl.BlockSpec((1, _CHUNK, _D),
                         lambda t: (jnp.maximum(t - 1, 0) // _NCHUNK,
                                    jnp.maximum(t - 1, 0) % _NCHUNK, 0)),
            pl.BlockSpec((1, _D, _CHUNK),
                         lambda t: (jnp.minimum(t, _T - 1) // _NCHUNK, 0,
                                    jnp.minimum(t, _T - 1) % _NCHUNK)),
            pl.BlockSpec((1, _CHUNK), lambda t: (0, 0)),
        ],
        out_specs=(
            pl.BlockSpec((n, _D), lambda t: (0, 0)),
            pl.BlockSpec((1, 1), lambda t: (0, 0)),
        ),
        out_shape=(
            jax.ShapeDtypeStruct((n, _D), jnp.float32),
            jax.ShapeDtypeStruct((1, 1), jnp.float32),
        ),
        scratch_shapes=[
            pltpu.VMEM((2, n, _CHUNK), jnp.float32),
            pltpu.VMEM((n, 1), jnp.float32),
            pltpu.VMEM((n, 3 * _D), jnp.float32),
            pltpu.VMEM((n, _D), jnp.float32),
            pltpu.VMEM((n, _D), jnp.float32),
            pltpu.VMEM((1, 1), jnp.float32),
        ],
    )(z_flat, codebooks, cbt, iota_row)
    return zq.reshape(orig_shape), loss[0, 0]


# speedup vs baseline: 1.1227x; 1.0026x over previous
import jax
import jax.numpy as jnp
from jax.experimental import pallas as pl
from jax.experimental.pallas import tpu as pltpu

_S = 4
_K = 8192
_D = 32
_CHUNK = 2048
_NCHUNK = _K // _CHUNK
_T = _S * _NCHUNK


def _rq_body(z_ref, cb_ref, cbt_ref, iota_ref, zq_ref, loss_ref,
             dbuf_ref, best_ref, qbest_ref, res_ref, recon_ref, lacc_ref):
    t = pl.program_id(0)
    p = t - 1

    @pl.when(t == 0)
    def _init():
        res_ref[...] = z_ref[...]
        recon_ref[...] = jnp.zeros_like(recon_ref)
        lacc_ref[...] = jnp.zeros_like(lacc_ref)

    @pl.when(t > 0)
    def _consume():
        dists = dbuf_ref[(t - 1) % 2]
        ib = iota_ref[...]
        nl = 128
        nj = _CHUNK // nl
        run_min = dists[:, 0:nl]
        run_idx = jnp.zeros_like(run_min)
        for j in range(1, nj):
            d_j = dists[:, j * nl:(j + 1) * nl]
            lt = d_j < run_min
            run_min = jnp.where(lt, d_j, run_min)
            run_idx = jnp.where(lt, float(j), run_idx)
        m = jnp.min(run_min, axis=1, keepdims=True)
        comb = run_idx * float(nl) + ib[:, 0:nl]
        li = jnp.min(jnp.where(run_min == m, comb, float(_CHUNK)),
                     axis=1, keepdims=True)
        onehot = (ib == li).astype(jnp.bfloat16)

        cb = cb_ref[0]
        hi = cb.astype(jnp.bfloat16)
        r1 = cb - hi.astype(jnp.float32)
        mid = r1.astype(jnp.bfloat16)
        lo = (r1 - mid.astype(jnp.float32)).astype(jnp.bfloat16)
        csplit = jnp.concatenate([hi, mid, lo], axis=1)
        qc3 = jax.lax.dot_general(
            onehot, csplit, (((1,), (0,)), ((), ())),
            preferred_element_type=jnp.float32)

        first = p % _NCHUNK == 0
        upd = jnp.logical_or(m < best_ref[...], first)
        best_ref[...] = jnp.where(upd, m, best_ref[...])
        qbest_ref[...] = jnp.where(upd, qc3, qbest_ref[...])

    @pl.when(jnp.logical_and(t > 0, p % _NCHUNK == _NCHUNK - 1))
    def _stage_fin():
        qb = qbest_ref[...]
        q = (qb[:, :_D] + qb[:, _D:2 * _D]) + qb[:, 2 * _D:]
        err = q - res_ref[...]
        sq = jnp.sum(err * err, axis=1, keepdims=True)
        lacc_ref[...] += jnp.sum(sq, axis=0, keepdims=True) / (err.shape[0] * _D)
        recon_ref[...] += q
        res_ref[...] = res_ref[...] - q

    @pl.when(t == _T)
    def _fin():
        z = z_ref[...]
        zq_ref[...] = z + (recon_ref[...] - z)
        loss_ref[...] = lacc_ref[...] * 1.25

    @pl.when(t < _T)
    def _produce():
        residual = res_ref[...]
        r2 = jnp.sum(residual * residual, axis=1, keepdims=True)
        rb2 = (residual + residual).astype(jnp.bfloat16)
        cbt = cbt_ref[0]
        c2 = jnp.sum(cbt * cbt, axis=0, keepdims=True)
        dots2 = jax.lax.dot_general(
            rb2, cbt.astype(jnp.bfloat16), (((1,), (0,)), ((), ())),
            preferred_element_type=jnp.float32)
        dbuf_ref[t % 2] = (r2 - dots2) + c2


def kernel(z, codebooks):
    orig_shape = z.shape
    z_flat = z.reshape(-1, _D)
    n = z_flat.shape[0]
    cbt = codebooks.transpose(0, 2, 1)
    iota_row = jnp.arange(_CHUNK, dtype=jnp.float32).reshape(1, _CHUNK)
    zq, loss = pl.pallas_call(
        _rq_body,
        grid=(_T + 1,),
        in_specs=[
            pl.BlockSpec((n, _D), lambda t: (0, 0)),
            pl.BlockSpec((1, _CHUNK, _D),
                         lambda t: (jnp.maximum(t - 1, 0) // _NCHUNK,
                                    jnp.maximum(t - 1, 0) % _NCHUNK, 0)),
            pl.BlockSpec((1, _D, _CHUNK),
                         lambda t: (jnp.minimum(t, _T - 1) // _NCHUNK, 0,
                                    jnp.minimum(t, _T - 1) % _NCHUNK)),
            pl.BlockSpec((1, _CHUNK), lambda t: (0, 0)),
        ],
        out_specs=(
            pl.BlockSpec((n, _D), lambda t: (0, 0)),
            pl.BlockSpec((1, 1), lambda t: (0, 0)),
        ),
        out_shape=(
            jax.ShapeDtypeStruct((n, _D), jnp.float32),
            jax.ShapeDtypeStruct((1, 1), jnp.float32),
        ),
        scratch_shapes=[
            pltpu.VMEM((2, n, _CHUNK), jnp.float32),
            pltpu.VMEM((n, 1), jnp.float32),
            pltpu.VMEM((n, 3 * _D), jnp.float32),
            pltpu.VMEM((n, _D), jnp.float32),
            pltpu.VMEM((n, _D), jnp.float32),
            pltpu.VMEM((1, 1), jnp.float32),
        ],
    )(z_flat, codebooks, cbt, iota_row)
    return zq.reshape(orig_shape), loss[0, 0]
